# baseline (speedup 1.0000x reference)
import functools

import jax
import jax.numpy as jnp
from jax import lax
from jax.experimental import pallas as pl
from jax.experimental.pallas import tpu as pltpu
from jax.experimental.pallas import tpu_sc as plsc

_NC = 2
_NS = 16
_NW = _NC * _NS
_L = 16


def _sc_gather_cols(t2, tails, idx_flat, coef_flat, n_rows, v, b, d):
  rows_per_w = n_rows // _NW
  nvec = b // _L
  nf = n_rows // d
  v_al = (v // 128) * 128
  tail = v - v_al
  sz_a = (v_al // 256) * 128
  sz_b = v_al - sz_a
  mesh = plsc.VectorSubcoreMesh(core_axis_name="c", subcore_axis_name="s")

  @functools.partial(
      pl.kernel,
      out_type=jax.ShapeDtypeStruct((nf, d * b), jnp.float32),
      mesh=mesh,
      scratch_types=[
          pltpu.VMEM((sz_a,), jnp.float32),
          pltpu.VMEM((sz_b + 128,), jnp.float32),
          pltpu.VMEM((2, b), jnp.int32),
          pltpu.VMEM((2, b), jnp.float32),
          pltpu.VMEM((2, b), jnp.float32),
          pltpu.SemaphoreType.DMA,
          pltpu.SemaphoreType.DMA,
          pltpu.SemaphoreType.DMA,
      ],
      compiler_params=pltpu.CompilerParams(needs_layout_passes=False),
  )
  def gather_kernel(t2_hbm, tails_hbm, idx_hbm, coef_hbm, out_hbm, buf_a,
                    buf_b, idxbuf, coefbuf, outbuf, asem, bsem, osem):
    sid = lax.axis_index("s")
    wid = sid * _NC + lax.axis_index("c")
    r0 = wid * rows_per_w
    j0 = r0 // d
    j1 = jnp.minimum(j0 + 1, nf - 1)

    def copy_a(r):
      return pltpu.make_async_copy(t2_hbm.at[r].at[pl.ds(0, sz_a)], buf_a,
                                   asem)

    def copy_b(r):
      return pltpu.make_async_copy(t2_hbm.at[r].at[pl.ds(sz_a, sz_b)],
                                   buf_b.at[pl.ds(0, sz_b)], bsem)

    def copy_t(r):
      return pltpu.make_async_copy(tails_hbm.at[r],
                                   buf_b.at[pl.ds(sz_b, 128)], bsem)

    pf = [
        pltpu.make_async_copy(idx_hbm.at[pl.ds(j0 * b, b)], idxbuf.at[0],
                              osem),
        pltpu.make_async_copy(idx_hbm.at[pl.ds(j1 * b, b)], idxbuf.at[1],
                              osem),
        pltpu.make_async_copy(coef_hbm.at[pl.ds(j0 * b, b)], coefbuf.at[0],
                              osem),
        pltpu.make_async_copy(coef_hbm.at[pl.ds(j1 * b, b)], coefbuf.at[1],
                              osem),
    ]
    for cp in pf:
      cp.start()
    copy_a(r0).start()
    copy_b(r0).start()
    copy_t(r0).start()
    for cp in pf:
      cp.wait()

    def row_body(k, carry):
      r = r0 + k
      j = r // d
      jl = j - j0
      kb = k % 2
      dst = out_hbm.at[j].at[pl.ds(pl.multiple_of((r % d) * b, b), b)]

      @pl.when(k >= 2)
      def _drain():
        pltpu.make_async_copy(outbuf.at[0], dst, osem).wait()

      copy_a(r).wait()

      @plsc.parallel_loop(0, nvec, 1, unroll=8)
      def g_a(i):
        off = pl.multiple_of(i * _L, _L)
        iv = idxbuf[jl, pl.ds(off, _L)]
        m = iv < sz_a
        ivc = jnp.where(m, iv, 0)
        vals = plsc.load_gather(buf_a, [ivc])
        outbuf[kb, pl.ds(off, _L)] = vals * coefbuf[jl, pl.ds(off, _L)]

      @pl.when(k + 1 < rows_per_w)
      def _next_a():
        copy_a(r + 1).start()

      copy_b(r).wait()
      copy_t(r).wait()

      @plsc.parallel_loop(0, nvec, 1, unroll=8)
      def g_b(i):
        off = pl.multiple_of(i * _L, _L)
        iv = idxbuf[jl, pl.ds(off, _L)]
        ivb = iv - sz_a
        m = iv >= sz_a
        ivc = jnp.where(m, ivb, 0)
        vals = plsc.load_gather(buf_b, [ivc])
        prev = outbuf[kb, pl.ds(off, _L)]
        outbuf[kb, pl.ds(off, _L)] = jnp.where(
            m, vals * coefbuf[jl, pl.ds(off, _L)], prev)

      @pl.when(k + 1 < rows_per_w)
      def _next_b():
        copy_b(r + 1).start()
        copy_t(r + 1).start()

      pltpu.make_async_copy(outbuf.at[kb], dst, osem).start()
      return carry

    lax.fori_loop(0, rows_per_w, row_body, 0)
    dst0 = out_hbm.at[j0].at[pl.ds(0, b)]
    pltpu.make_async_copy(outbuf.at[0], dst0, osem).wait()
    pltpu.make_async_copy(outbuf.at[1], dst0, osem).wait()

  return gather_kernel(t2, tails, idx_flat, coef_flat)


def _dense_body(x_ref, ws_ref, w0_ref, out_ref):
  f = ws_ref.shape[2]
  x = x_ref[...]
  y = jnp.dot(ws_ref[0], x, preferred_element_type=jnp.float32)
  part = 0.5 * jnp.sum(x * y[:f, :], axis=0) + y[f, :]

  @pl.when(pl.program_id(0) == 0)
  def _init():
    out_ref[...] = jnp.full_like(out_ref, w0_ref[0, 0])

  out_ref[...] += part


def _tc_dense(x2, wstack, w0, f, d, b):
  grid = (d,)
  return pl.pallas_call(
      _dense_body,
      grid=grid,
      in_specs=[
          pl.BlockSpec((f, b), lambda i: (0, i)),
          pl.BlockSpec((1, f + 1, f), lambda i: (i, 0, 0)),
          pl.BlockSpec(memory_space=pltpu.SMEM),
      ],
      out_specs=pl.BlockSpec((b,), lambda i: (0,)),
      out_shape=jax.ShapeDtypeStruct((b,), jnp.float32),
  )(x2, wstack, w0)


@jax.jit
def kernel(index, coef, tables, field_emb, W, w0):
  b, f = index.shape
  _, v, d = tables.shape
  fd = f * d

  t2 = tables.transpose(0, 2, 1).reshape(fd, v)
  tails = jnp.pad(t2[:, (v // 128) * 128:], ((0, 0), (0, 128 - v % 128)))
  idx_flat = index.T.reshape(-1)
  coef_flat = coef.T.reshape(-1)

  x2 = _sc_gather_cols(t2, tails, idx_flat, coef_flat, fd, v, b, d)

  sym_w = (W + W.T) * 0.5
  wp = sym_w * (1.0 - jnp.eye(f, dtype=jnp.float32))
  wstack = jnp.concatenate(
      [jnp.broadcast_to(wp[None], (d, f, f)), field_emb.T[:, None, :]],
      axis=1)
  w0s = w0.reshape(1, 1)

  out = _tc_dense(x2, wstack, w0s, f, d, b)
  cfe = x2.reshape(f, d, b).transpose(2, 0, 1)
  return out, cfe


# speedup vs baseline: 1.1274x; 1.1274x over previous
import functools

import jax
import jax.numpy as jnp
from jax import lax
from jax.experimental import pallas as pl
from jax.experimental.pallas import tpu as pltpu
from jax.experimental.pallas import tpu_sc as plsc

_NC = 2
_NS = 16
_NW = _NC * _NS
_L = 16


def _sc_gather_cols(t2, tails, idx_flat, coef_flat, n_rows, v, b, d):
  rows_per_w = n_rows // _NW
  nvec = b // _L
  nf = n_rows // d
  v_al = (v // 128) * 128
  tail = v - v_al
  sz_a = (v_al // 256) * 128
  sz_b = v_al - sz_a
  mesh = plsc.VectorSubcoreMesh(core_axis_name="c", subcore_axis_name="s")

  @functools.partial(
      pl.kernel,
      out_type=jax.ShapeDtypeStruct((n_rows, b), jnp.float32),
      mesh=mesh,
      scratch_types=[
          pltpu.VMEM((sz_a,), jnp.float32),
          pltpu.VMEM((sz_b + 128,), jnp.float32),
          pltpu.VMEM((2, b), jnp.int32),
          pltpu.VMEM((2, b), jnp.float32),
          pltpu.VMEM((2, b), jnp.float32),
          pltpu.SemaphoreType.DMA,
          pltpu.SemaphoreType.DMA,
          pltpu.SemaphoreType.DMA,
      ],
      compiler_params=pltpu.CompilerParams(needs_layout_passes=False),
  )
  def gather_kernel(t2_hbm, tails_hbm, idx_hbm, coef_hbm, out_hbm, buf_a,
                    buf_b, idxbuf, coefbuf, outbuf, asem, bsem, osem):
    sid = lax.axis_index("s")
    wid = sid * _NC + lax.axis_index("c")
    r0 = wid * rows_per_w
    j0 = r0 // d
    j1 = jnp.minimum(j0 + 1, nf - 1)

    def copy_a(r):
      return pltpu.make_async_copy(t2_hbm.at[r].at[pl.ds(0, sz_a)], buf_a,
                                   asem)

    def copy_b(r):
      return pltpu.make_async_copy(t2_hbm.at[r].at[pl.ds(sz_a, sz_b)],
                                   buf_b.at[pl.ds(0, sz_b)], bsem)

    def copy_t(r):
      return pltpu.make_async_copy(tails_hbm.at[r],
                                   buf_b.at[pl.ds(sz_b, 128)], bsem)

    pf = [
        pltpu.make_async_copy(idx_hbm.at[pl.ds(j0 * b, b)], idxbuf.at[0],
                              osem),
        pltpu.make_async_copy(idx_hbm.at[pl.ds(j1 * b, b)], idxbuf.at[1],
                              osem),
        pltpu.make_async_copy(coef_hbm.at[pl.ds(j0 * b, b)], coefbuf.at[0],
                              osem),
        pltpu.make_async_copy(coef_hbm.at[pl.ds(j1 * b, b)], coefbuf.at[1],
                              osem),
    ]
    for cp in pf:
      cp.start()
    copy_a(r0).start()
    copy_b(r0).start()
    copy_t(r0).start()
    for cp in pf:
      cp.wait()

    def row_body(k, carry):
      r = r0 + k
      jl = r // d - j0
      kb = k % 2
      dst = out_hbm.at[r]

      @pl.when(k >= 2)
      def _drain():
        pltpu.make_async_copy(outbuf.at[0], dst, osem).wait()

      copy_a(r).wait()

      @plsc.parallel_loop(0, nvec, 1, unroll=8)
      def g_a(i):
        off = pl.multiple_of(i * _L, _L)
        iv = idxbuf[jl, pl.ds(off, _L)]
        m = iv < sz_a
        ivc = jnp.where(m, iv, 0)
        vals = plsc.load_gather(buf_a, [ivc])
        outbuf[kb, pl.ds(off, _L)] = vals * coefbuf[jl, pl.ds(off, _L)]

      @pl.when(k + 1 < rows_per_w)
      def _next_a():
        copy_a(r + 1).start()

      copy_b(r).wait()
      copy_t(r).wait()

      @plsc.parallel_loop(0, nvec, 1, unroll=8)
      def g_b(i):
        off = pl.multiple_of(i * _L, _L)
        iv = idxbuf[jl, pl.ds(off, _L)]
        ivb = iv - sz_a
        m = iv >= sz_a
        ivc = jnp.where(m, ivb, 0)
        vals = plsc.load_gather(buf_b, [ivc])
        prev = outbuf[kb, pl.ds(off, _L)]
        outbuf[kb, pl.ds(off, _L)] = jnp.where(
            m, vals * coefbuf[jl, pl.ds(off, _L)], prev)

      @pl.when(k + 1 < rows_per_w)
      def _next_b():
        copy_b(r + 1).start()
        copy_t(r + 1).start()

      pltpu.make_async_copy(outbuf.at[kb], dst, osem).start()
      return carry

    lax.fori_loop(0, rows_per_w, row_body, 0)
    pltpu.make_async_copy(outbuf.at[0], out_hbm.at[r0], osem).wait()
    pltpu.make_async_copy(outbuf.at[1], out_hbm.at[r0], osem).wait()

  return gather_kernel(t2, tails, idx_flat, coef_flat)


def _dense_body(x_ref, wk_ref, femb_ref, w0_ref, out_ref):
  x = x_ref[...]
  y = jnp.dot(wk_ref[...], x, preferred_element_type=jnp.float32)
  ffi = jnp.sum(x * femb_ref[...], axis=0)
  inter = 0.5 * jnp.sum(x * y, axis=0)
  out_ref[...] = w0_ref[0, 0] + ffi + inter


def _tc_dense(x_cols, wkron, femb_col, w0):
  fd, b = x_cols.shape
  bb = 1024
  grid = (b // bb,)
  return pl.pallas_call(
      _dense_body,
      grid=grid,
      in_specs=[
          pl.BlockSpec((fd, bb), lambda i: (0, i)),
          pl.BlockSpec((fd, fd), lambda i: (0, 0)),
          pl.BlockSpec((fd, 1), lambda i: (0, 0)),
          pl.BlockSpec(memory_space=pltpu.SMEM),
      ],
      out_specs=pl.BlockSpec((bb,), lambda i: (i,)),
      out_shape=jax.ShapeDtypeStruct((b,), jnp.float32),
  )(x_cols, wkron, femb_col, w0)


@jax.jit
def kernel(index, coef, tables, field_emb, W, w0):
  b, f = index.shape
  _, v, d = tables.shape
  fd = f * d

  t2 = tables.transpose(0, 2, 1).reshape(fd, v)
  tails = jnp.pad(t2[:, (v // 128) * 128:], ((0, 0), (0, 128 - v % 128)))
  idx_flat = index.T.reshape(-1)
  coef_flat = coef.T.reshape(-1)

  x_cols = _sc_gather_cols(t2, tails, idx_flat, coef_flat, fd, v, b, d)

  sym_w = (W + W.T) * 0.5
  wp = sym_w * (1.0 - jnp.eye(f, dtype=jnp.float32))
  wkron = jnp.kron(wp, jnp.eye(d, dtype=jnp.float32))
  femb_col = field_emb.reshape(fd, 1)
  w0s = w0.reshape(1, 1)

  out = _tc_dense(x_cols, wkron, femb_col, w0s)
  cfe = x_cols.reshape(f, d, b).transpose(2, 0, 1)
  return out, cfe


# speedup vs baseline: 1.1593x; 1.0284x over previous
import functools

import jax
import jax.numpy as jnp
from jax import lax
from jax.experimental import pallas as pl
from jax.experimental.pallas import tpu as pltpu
from jax.experimental.pallas import tpu_sc as plsc

_NC = 2
_NS = 16
_NW = _NC * _NS
_L = 16


def _sc_gather_cols(t2, tails, idx_t, coef_t, n_rows, v, b, d):
  rows_per_w = n_rows // _NW
  nvec = b // _L
  nf = n_rows // d
  v_al = (v // 128) * 128
  tail = v - v_al
  sz_a = (v_al // 256) * 128
  sz_b = v_al - sz_a
  mesh = plsc.VectorSubcoreMesh(core_axis_name="c", subcore_axis_name="s")

  @functools.partial(
      pl.kernel,
      out_type=jax.ShapeDtypeStruct((n_rows, b), jnp.float32),
      mesh=mesh,
      scratch_types=[
          pltpu.VMEM((sz_a,), jnp.float32),
          pltpu.VMEM((sz_b + 128,), jnp.float32),
          pltpu.VMEM((2, b), jnp.int32),
          pltpu.VMEM((2, b), jnp.float32),
          pltpu.VMEM((2, b), jnp.float32),
          pltpu.SemaphoreType.DMA,
          pltpu.SemaphoreType.DMA,
          pltpu.SemaphoreType.DMA,
      ],
      compiler_params=pltpu.CompilerParams(needs_layout_passes=False),
  )
  def gather_kernel(t2_hbm, tails_hbm, idx_hbm, coef_hbm, out_hbm, buf_a,
                    buf_b, idxbuf, coefbuf, outbuf, asem, bsem, osem):
    sid = lax.axis_index("s")
    wid = sid * _NC + lax.axis_index("c")
    r0 = wid * rows_per_w
    j0 = r0 // d
    j1 = jnp.minimum(j0 + 1, nf - 1)

    def copy_a(r):
      return pltpu.make_async_copy(t2_hbm.at[r].at[pl.ds(0, sz_a)], buf_a,
                                   asem)

    def copy_b(r):
      return pltpu.make_async_copy(t2_hbm.at[r].at[pl.ds(sz_a, sz_b)],
                                   buf_b.at[pl.ds(0, sz_b)], bsem)

    def copy_t(r):
      return pltpu.make_async_copy(tails_hbm.at[r],
                                   buf_b.at[pl.ds(sz_b, 128)], bsem)

    pf = [
        pltpu.make_async_copy(idx_hbm.at[j0], idxbuf.at[0], osem),
        pltpu.make_async_copy(idx_hbm.at[j1], idxbuf.at[1], osem),
        pltpu.make_async_copy(coef_hbm.at[j0], coefbuf.at[0], osem),
        pltpu.make_async_copy(coef_hbm.at[j1], coefbuf.at[1], osem),
    ]
    for cp in pf:
      cp.start()
    copy_a(r0).start()
    copy_b(r0).start()
    copy_t(r0).start()
    for cp in pf:
      cp.wait()

    def row_body(k, carry):
      r = r0 + k
      jl = r // d - j0
      kb = k % 2
      dst = out_hbm.at[r]

      @pl.when(k >= 2)
      def _drain():
        pltpu.make_async_copy(outbuf.at[0], dst, osem).wait()

      copy_a(r).wait()

      @plsc.parallel_loop(0, nvec, 1, unroll=8)
      def g_a(i):
        off = pl.multiple_of(i * _L, _L)
        iv = idxbuf[jl, pl.ds(off, _L)]
        m = iv < sz_a
        ivc = jnp.where(m, iv, 0)
        vals = plsc.load_gather(buf_a, [ivc])
        outbuf[kb, pl.ds(off, _L)] = vals * coefbuf[jl, pl.ds(off, _L)]

      @pl.when(k + 1 < rows_per_w)
      def _next_a():
        copy_a(r + 1).start()

      copy_b(r).wait()
      copy_t(r).wait()

      @plsc.parallel_loop(0, nvec, 1, unroll=8)
      def g_b(i):
        off = pl.multiple_of(i * _L, _L)
        iv = idxbuf[jl, pl.ds(off, _L)]
        ivb = iv - sz_a
        m = iv >= sz_a
        ivc = jnp.where(m, ivb, 0)
        vals = plsc.load_gather(buf_b, [ivc])
        prev = outbuf[kb, pl.ds(off, _L)]
        outbuf[kb, pl.ds(off, _L)] = jnp.where(
            m, vals * coefbuf[jl, pl.ds(off, _L)], prev)

      @pl.when(k + 1 < rows_per_w)
      def _next_b():
        copy_b(r + 1).start()
        copy_t(r + 1).start()

      pltpu.make_async_copy(outbuf.at[kb], dst, osem).start()
      return carry

    lax.fori_loop(0, rows_per_w, row_body, 0)
    pltpu.make_async_copy(outbuf.at[0], out_hbm.at[r0], osem).wait()
    pltpu.make_async_copy(outbuf.at[1], out_hbm.at[r0], osem).wait()

  return gather_kernel(t2, tails, idx_t, coef_t)


def _dense_body(x_ref, wk_ref, femb_ref, w0_ref, out_ref):
  x = x_ref[...]
  y = jnp.dot(wk_ref[...], x, preferred_element_type=jnp.float32)
  ffi = jnp.sum(x * femb_ref[...], axis=0)
  inter = 0.5 * jnp.sum(x * y, axis=0)
  out_ref[...] = w0_ref[0, 0] + ffi + inter


def _tc_dense(x_cols, wkron, femb_col, w0):
  fd, b = x_cols.shape
  bb = 1024
  grid = (b // bb,)
  return pl.pallas_call(
      _dense_body,
      grid=grid,
      in_specs=[
          pl.BlockSpec((fd, bb), lambda i: (0, i)),
          pl.BlockSpec((fd, fd), lambda i: (0, 0)),
          pl.BlockSpec((fd, 1), lambda i: (0, 0)),
          pl.BlockSpec(memory_space=pltpu.SMEM),
      ],
      out_specs=pl.BlockSpec((bb,), lambda i: (i,)),
      out_shape=jax.ShapeDtypeStruct((b,), jnp.float32),
  )(x_cols, wkron, femb_col, w0)


@jax.jit
def kernel(index, coef, tables, field_emb, W, w0):
  b, f = index.shape
  _, v, d = tables.shape
  fd = f * d

  t2 = tables.transpose(0, 2, 1).reshape(fd, v)
  tails = jnp.pad(t2[:, (v // 128) * 128:], ((0, 0), (0, 128 - v % 128)))
  idx_t = index.T
  coef_t = coef.T

  x_cols = _sc_gather_cols(t2, tails, idx_t, coef_t, fd, v, b, d)

  sym_w = (W + W.T) * 0.5
  wp = sym_w * (1.0 - jnp.eye(f, dtype=jnp.float32))
  wkron = jnp.kron(wp, jnp.eye(d, dtype=jnp.float32))
  femb_col = field_emb.reshape(fd, 1)
  w0s = w0.reshape(1, 1)

  out = _tc_dense(x_cols, wkron, femb_col, w0s)
  cfe = x_cols.reshape(f, d, b).transpose(2, 0, 1)
  return out, cfe


# speedup vs baseline: 1.1732x; 1.0120x over previous
import functools

import jax
import jax.numpy as jnp
from jax import lax
from jax.experimental import pallas as pl
from jax.experimental.pallas import tpu as pltpu
from jax.experimental.pallas import tpu_sc as plsc

_NC = 2
_NS = 16
_NW = _NC * _NS
_L = 16


def _sc_gather_cols(t2, tails, idx_t, coef_t, n_rows, v, b, d):
  rows_per_w = n_rows // _NW
  nvec = b // _L
  nf = n_rows // d
  v_al = (v // 128) * 128
  tail = v - v_al
  sz_a = (v_al // 256) * 128
  sz_b = v_al - sz_a
  mesh = plsc.VectorSubcoreMesh(core_axis_name="c", subcore_axis_name="s")

  @functools.partial(
      pl.kernel,
      out_type=jax.ShapeDtypeStruct((n_rows, b), jnp.float32),
      mesh=mesh,
      scratch_types=[
          pltpu.VMEM((sz_a,), jnp.float32),
          pltpu.VMEM((sz_b + 128,), jnp.float32),
          pltpu.VMEM((2, b), jnp.int32),
          pltpu.VMEM((2, b), jnp.float32),
          pltpu.VMEM((2, b), jnp.float32),
          pltpu.SemaphoreType.DMA,
          pltpu.SemaphoreType.DMA,
          pltpu.SemaphoreType.DMA,
      ],
      compiler_params=pltpu.CompilerParams(needs_layout_passes=False),
  )
  def gather_kernel(t2_hbm, tails_hbm, idx_hbm, coef_hbm, out_hbm, buf_a,
                    buf_b, idxbuf, coefbuf, outbuf, asem, bsem, osem):
    sid = lax.axis_index("s")
    wid = sid * _NC + lax.axis_index("c")
    r0 = wid * rows_per_w
    j0 = r0 // d
    j1 = jnp.minimum(j0 + 1, nf - 1)

    def copy_a(r):
      return pltpu.make_async_copy(t2_hbm.at[r].at[pl.ds(0, sz_a)], buf_a,
                                   asem)

    def copy_b(r):
      return pltpu.make_async_copy(t2_hbm.at[r].at[pl.ds(sz_a, sz_b)],
                                   buf_b.at[pl.ds(0, sz_b)], bsem)

    def copy_t(r):
      return pltpu.make_async_copy(tails_hbm.at[r],
                                   buf_b.at[pl.ds(sz_b, 128)], bsem)

    pf = [
        pltpu.make_async_copy(idx_hbm.at[j0], idxbuf.at[0], osem),
        pltpu.make_async_copy(idx_hbm.at[j1], idxbuf.at[1], osem),
        pltpu.make_async_copy(coef_hbm.at[j0], coefbuf.at[0], osem),
        pltpu.make_async_copy(coef_hbm.at[j1], coefbuf.at[1], osem),
    ]
    for cp in pf:
      cp.start()
    copy_a(r0).start()
    copy_b(r0).start()
    copy_t(r0).start()
    for cp in pf:
      cp.wait()

    def row_body(k, carry):
      r = r0 + k
      jl = r // d - j0
      kb = k % 2
      dst = out_hbm.at[r]

      @pl.when(k >= 2)
      def _drain():
        pltpu.make_async_copy(outbuf.at[0], dst, osem).wait()

      copy_a(r).wait()

      @plsc.parallel_loop(0, nvec, 1, unroll=8)
      def g_a(i):
        off = pl.multiple_of(i * _L, _L)
        iv = idxbuf[jl, pl.ds(off, _L)]
        m = iv < sz_a
        ivc = jnp.where(m, iv, 0)
        vals = plsc.load_gather(buf_a, [ivc])
        outbuf[kb, pl.ds(off, _L)] = vals * coefbuf[jl, pl.ds(off, _L)]

      @pl.when(k + 1 < rows_per_w)
      def _next_a():
        copy_a(r + 1).start()

      copy_b(r).wait()
      copy_t(r).wait()

      @plsc.parallel_loop(0, nvec, 1, unroll=8)
      def g_b(i):
        off = pl.multiple_of(i * _L, _L)
        iv = idxbuf[jl, pl.ds(off, _L)]
        ivb = iv - sz_a
        m = iv >= sz_a
        ivc = jnp.where(m, ivb, 0)
        vals = plsc.load_gather(buf_b, [ivc])
        prev = outbuf[kb, pl.ds(off, _L)]
        outbuf[kb, pl.ds(off, _L)] = jnp.where(
            m, vals * coefbuf[jl, pl.ds(off, _L)], prev)

      @pl.when(k + 1 < rows_per_w)
      def _next_b():
        copy_b(r + 1).start()
        copy_t(r + 1).start()

      pltpu.make_async_copy(outbuf.at[kb], dst, osem).start()
      return carry

    lax.fori_loop(0, rows_per_w, row_body, 0)
    pltpu.make_async_copy(outbuf.at[0], out_hbm.at[r0], osem).wait()
    pltpu.make_async_copy(outbuf.at[1], out_hbm.at[r0], osem).wait()

  return gather_kernel(t2, tails, idx_t, coef_t)


def _dense_body(x_ref, wk_ref, femb_ref, w0_ref, out_ref):
  x = x_ref[...]
  y = jnp.dot(wk_ref[...], x, preferred_element_type=jnp.float32)
  ffi = jnp.sum(x * femb_ref[...], axis=0)
  inter = 0.5 * jnp.sum(x * y, axis=0)
  out_ref[...] = w0_ref[0, 0] + ffi + inter


def _tc_dense(x_cols, wkron, femb_col, w0):
  fd, b = x_cols.shape
  bb = 2048
  grid = (b // bb,)
  return pl.pallas_call(
      _dense_body,
      grid=grid,
      in_specs=[
          pl.BlockSpec((fd, bb), lambda i: (0, i)),
          pl.BlockSpec((fd, fd), lambda i: (0, 0)),
          pl.BlockSpec((fd, 1), lambda i: (0, 0)),
          pl.BlockSpec(memory_space=pltpu.SMEM),
      ],
      out_specs=pl.BlockSpec((bb,), lambda i: (i,)),
      out_shape=jax.ShapeDtypeStruct((b,), jnp.float32),
  )(x_cols, wkron, femb_col, w0)


@jax.jit
def kernel(index, coef, tables, field_emb, W, w0):
  b, f = index.shape
  _, v, d = tables.shape
  fd = f * d

  t2 = tables.transpose(0, 2, 1).reshape(fd, v)
  tails = jnp.pad(t2[:, (v // 128) * 128:], ((0, 0), (0, 128 - v % 128)))
  idx_t = index.T
  coef_t = coef.T

  x_cols = _sc_gather_cols(t2, tails, idx_t, coef_t, fd, v, b, d)

  sym_w = (W + W.T) * 0.5
  wp = sym_w * (1.0 - jnp.eye(f, dtype=jnp.float32))
  wkron = jnp.kron(wp, jnp.eye(d, dtype=jnp.float32))
  femb_col = field_emb.reshape(fd, 1)
  w0s = w0.reshape(1, 1)

  out = _tc_dense(x_cols, wkron, femb_col, w0s)
  cfe = x_cols.reshape(f, d, b).transpose(2, 0, 1)
  return out, cfe
